# baseline (speedup 1.0000x reference)
import jax
import jax.numpy as jnp
from jax import lax
from jax.experimental import pallas as pl
from jax.experimental.pallas import tpu as pltpu


def _pick_bm(n):
    best = None
    for bm in range(8, 1001, 8):
        if n % bm == 0:
            best = bm
    return best if best is not None else n


def _pick_bk(ncols_main):
    best = 128
    for bk in range(128, 2049, 128):
        if ncols_main % bk == 0:
            best = bk
    return best


def kernel(x, adj, W1, b1, W2, b2, lin_w, lin_b):
    n, nfeat = x.shape
    h1 = W1.shape[1]
    h2 = W2.shape[1]
    ncls = lin_w.shape[1]
    ncls_pad = ((ncls + 127) // 128) * 128

    bm = _pick_bm(n)
    nblk = n // bm
    ncols_main = (n // 128) * 128
    rem = n - ncols_main
    bk = _pick_bk(ncols_main)
    nk = ncols_main // bk

    SW = ((max(rem, 1) + 7) // 8) * 8
    n_pad = max(n, ncols_main + SW)

    strip = lax.slice(adj, (0, ncols_main), (n, n))
    if rem < SW:
        strip = jnp.pad(strip, ((0, 0), (0, SW - rem)))
    xp = jnp.pad(x, ((0, n_pad - n), (0, 0))) if n_pad > n else x
    s2_tail = n_pad - n

    b1r = b1.reshape(1, h1)
    b2r = b2.reshape(1, h2)
    lw = jnp.pad(lin_w, ((0, 0), (0, ncls_pad - ncls)))
    lb = jnp.pad(lin_b, (0, ncls_pad - ncls)).reshape(1, ncls_pad)

    def body(adj_ref, strip_ref, x_ref, w1_ref, b1_ref, w2_ref, b2_ref,
             lw_ref, lb_ref, out_ref, s2_ref, oacc_ref, tacc_ref):
        p = pl.program_id(0)
        m = pl.program_id(1)
        k = pl.program_id(2)

        t = adj_ref[...]
        mrows = pl.ds(m * bm, bm)
        krows = pl.ds(k * bk, bk)
        kf = (m * bm) // bk

        @pl.when(p == 0)
        def _phase0():
            prod = jnp.dot(t, x_ref[krows, :],
                           preferred_element_type=jnp.float32)

            @pl.when(k == 0)
            def _init():
                tacc_ref[...] = prod + jnp.dot(
                    strip_ref[...], x_ref[pl.ds(ncols_main, SW), :],
                    preferred_element_type=jnp.float32)
                oacc_ref[mrows, :] = jnp.zeros((bm, h2), jnp.float32)
                if s2_tail > 0:
                    @pl.when(m == 0)
                    def _zero_tail():
                        s2_ref[pl.ds(n, s2_tail), :] = jnp.zeros(
                            (s2_tail, h2), jnp.float32)

            @pl.when(k > 0)
            def _acc():
                tacc_ref[...] += prod

            @pl.when(k < kf)
            def _dual():
                oacc_ref[mrows, :] += jnp.dot(
                    t, s2_ref[krows, :], preferred_element_type=jnp.float32)

            @pl.when(k == nk - 1)
            def _epilogue1():
                z = jnp.dot(tacc_ref[...], w1_ref[...],
                            preferred_element_type=jnp.float32) + b1_ref[...]
                h = jnp.maximum(z, 0.1 * z)
                s2_ref[mrows, :] = jnp.dot(
                    h, w2_ref[...], preferred_element_type=jnp.float32)

        @pl.when(p == 1)
        def _phase1():
            @pl.when(k >= kf)
            def _upper():
                oacc_ref[mrows, :] += jnp.dot(
                    t, s2_ref[krows, :], preferred_element_type=jnp.float32)

            @pl.when(k == nk - 1)
            def _epilogue2():
                z = oacc_ref[mrows, :] + jnp.dot(
                    strip_ref[...], s2_ref[pl.ds(ncols_main, SW), :],
                    preferred_element_type=jnp.float32) + b2_ref[...]
                h = jnp.maximum(z, 0.1 * z)
                out_ref[...] = jnp.dot(
                    h, lw_ref[...], preferred_element_type=jnp.float32) \
                    + lb_ref[...]

    def adj_index(p, m, k):
        return (m, k)

    out = pl.pallas_call(
        body,
        grid=(2, nblk, nk),
        in_specs=[
            pl.BlockSpec((bm, bk), adj_index),
            pl.BlockSpec((bm, SW), lambda p, m, k: (m, 0)),
            pl.BlockSpec((n_pad, nfeat), lambda p, m, k: (0, 0)),
            pl.BlockSpec((nfeat, h1), lambda p, m, k: (0, 0)),
            pl.BlockSpec((1, h1), lambda p, m, k: (0, 0)),
            pl.BlockSpec((h1, h2), lambda p, m, k: (0, 0)),
            pl.BlockSpec((1, h2), lambda p, m, k: (0, 0)),
            pl.BlockSpec((h2, ncls_pad), lambda p, m, k: (0, 0)),
            pl.BlockSpec((1, ncls_pad), lambda p, m, k: (0, 0)),
        ],
        out_specs=pl.BlockSpec((bm, ncls_pad), lambda p, m, k: (p * m, 0)),
        out_shape=jax.ShapeDtypeStruct((n, ncls_pad), jnp.float32),
        scratch_shapes=[
            pltpu.VMEM((n_pad, h2), jnp.float32),
            pltpu.VMEM((n, h2), jnp.float32),
            pltpu.VMEM((bm, nfeat), jnp.float32),
        ],
        compiler_params=pltpu.CompilerParams(
            dimension_semantics=("arbitrary", "arbitrary", "arbitrary"),
        ),
    )(adj, strip, xp, W1, b1r, W2, b2r, lw, lb)

    return out[:, :ncls]


# speedup vs baseline: 1.2605x; 1.2605x over previous
import jax
import jax.numpy as jnp
from jax import lax
from jax.experimental import pallas as pl
from jax.experimental.pallas import tpu as pltpu


def _pick_bm(n):
    best = None
    for bm in range(8, 2001, 8):
        if n % bm == 0:
            best = bm
    return best if best is not None else n


def _pick_bk(ncols_main):
    best = 128
    for bk in range(128, 2049, 128):
        if ncols_main % bk == 0:
            best = bk
    return best


def kernel(x, adj, W1, b1, W2, b2, lin_w, lin_b):
    n, nfeat = x.shape
    h1 = W1.shape[1]
    h2 = W2.shape[1]
    ncls = lin_w.shape[1]
    ncls_pad = ((ncls + 127) // 128) * 128

    bm = _pick_bm(n)
    nblk = n // bm
    ncols_main = (n // 128) * 128
    rem = n - ncols_main
    bk = _pick_bk(ncols_main)
    nk = ncols_main // bk

    SW = ((max(rem, 1) + 7) // 8) * 8
    n_pad = max(n, ncols_main + SW)

    strip = lax.slice(adj, (0, ncols_main), (n, n))
    if rem < SW:
        strip = jnp.pad(strip, ((0, 0), (0, SW - rem)))
    xp = jnp.pad(x, ((0, n_pad - n), (0, 0))) if n_pad > n else x
    s2_tail = n_pad - n

    b1r = b1.reshape(1, h1)
    b2r = b2.reshape(1, h2)
    lw = jnp.pad(lin_w, ((0, 0), (0, ncls_pad - ncls)))
    lb = jnp.pad(lin_b, (0, ncls_pad - ncls)).reshape(1, ncls_pad)

    def body(adj_ref, strip_ref, x_ref, w1_ref, b1_ref, w2_ref, b2_ref,
             lw_ref, lb_ref, out_ref, s2_ref, oacc_ref, tacc_ref):
        p = pl.program_id(0)
        m = pl.program_id(1)
        k = pl.program_id(2)

        t = adj_ref[...]
        mrows = pl.ds(m * bm, bm)
        krows = pl.ds(k * bk, bk)
        kf = (m * bm) // bk

        @pl.when(p == 0)
        def _phase0():
            prod = jnp.dot(t, x_ref[krows, :],
                           preferred_element_type=jnp.float32)

            @pl.when(k == 0)
            def _init():
                tacc_ref[...] = prod + jnp.dot(
                    strip_ref[...], x_ref[pl.ds(ncols_main, SW), :],
                    preferred_element_type=jnp.float32)
                oacc_ref[mrows, :] = jnp.zeros((bm, h2), jnp.float32)
                if s2_tail > 0:
                    @pl.when(m == 0)
                    def _zero_tail():
                        s2_ref[pl.ds(n, s2_tail), :] = jnp.zeros(
                            (s2_tail, h2), jnp.float32)

            @pl.when(k > 0)
            def _acc():
                tacc_ref[...] += prod

            @pl.when(k < kf)
            def _dual():
                oacc_ref[mrows, :] += jnp.dot(
                    t, s2_ref[krows, :], preferred_element_type=jnp.float32)

            @pl.when(k == nk - 1)
            def _epilogue1():
                z = jnp.dot(tacc_ref[...], w1_ref[...],
                            preferred_element_type=jnp.float32) + b1_ref[...]
                h = jnp.maximum(z, 0.1 * z)
                s2_ref[mrows, :] = jnp.dot(
                    h, w2_ref[...], preferred_element_type=jnp.float32)

        @pl.when(p == 1)
        def _phase1():
            @pl.when(k >= kf)
            def _upper():
                oacc_ref[mrows, :] += jnp.dot(
                    t, s2_ref[krows, :], preferred_element_type=jnp.float32)

            @pl.when(k == nk - 1)
            def _epilogue2():
                z = oacc_ref[mrows, :] + jnp.dot(
                    strip_ref[...], s2_ref[pl.ds(ncols_main, SW), :],
                    preferred_element_type=jnp.float32) + b2_ref[...]
                h = jnp.maximum(z, 0.1 * z)
                out_ref[...] = jnp.dot(
                    h, lw_ref[...], preferred_element_type=jnp.float32) \
                    + lb_ref[...]

    def adj_index(p, m, k):
        kf = (m * bm) // bk
        return (m, jnp.where(p == 0, k, jnp.maximum(k, kf)))

    out = pl.pallas_call(
        body,
        grid=(2, nblk, nk),
        in_specs=[
            pl.BlockSpec((bm, bk), adj_index),
            pl.BlockSpec((bm, SW), lambda p, m, k: (m, 0)),
            pl.BlockSpec((n_pad, nfeat), lambda p, m, k: (0, 0)),
            pl.BlockSpec((nfeat, h1), lambda p, m, k: (0, 0)),
            pl.BlockSpec((1, h1), lambda p, m, k: (0, 0)),
            pl.BlockSpec((h1, h2), lambda p, m, k: (0, 0)),
            pl.BlockSpec((1, h2), lambda p, m, k: (0, 0)),
            pl.BlockSpec((h2, ncls_pad), lambda p, m, k: (0, 0)),
            pl.BlockSpec((1, ncls_pad), lambda p, m, k: (0, 0)),
        ],
        out_specs=pl.BlockSpec((bm, ncls_pad), lambda p, m, k: (p * m, 0)),
        out_shape=jax.ShapeDtypeStruct((n, ncls_pad), jnp.float32),
        scratch_shapes=[
            pltpu.VMEM((n_pad, h2), jnp.float32),
            pltpu.VMEM((n, h2), jnp.float32),
            pltpu.VMEM((bm, nfeat), jnp.float32),
        ],
        compiler_params=pltpu.CompilerParams(
            dimension_semantics=("arbitrary", "arbitrary", "arbitrary"),
            vmem_limit_bytes=64 * 1024 * 1024,
        ),
    )(adj, strip, xp, W1, b1r, W2, b2r, lw, lb)

    return out[:, :ncls]


# speedup vs baseline: 1.2692x; 1.0070x over previous
import jax
import jax.numpy as jnp
from jax import lax
from jax.experimental import pallas as pl
from jax.experimental.pallas import tpu as pltpu


def _pick_bm(n):
    best = None
    for bm in range(8, 2001, 8):
        if n % bm == 0:
            best = bm
    return best if best is not None else n


def _pick_bk(ncols_main):
    best = 128
    for bk in range(128, 2049, 128):
        if ncols_main % bk == 0:
            best = bk
    return best


def kernel(x, adj, W1, b1, W2, b2, lin_w, lin_b):
    n, nfeat = x.shape
    h1 = W1.shape[1]
    h2 = W2.shape[1]
    ncls = lin_w.shape[1]
    ncls_pad = ((ncls + 127) // 128) * 128

    bm = _pick_bm(n)
    nblk = n // bm
    ncols_main = (n // 128) * 128
    rem = n - ncols_main
    bk = _pick_bk(ncols_main)
    nk = ncols_main // bk

    bf16 = jnp.bfloat16

    SW = ((max(rem, 1) + 7) // 8) * 8
    n_pad = max(n, ncols_main + SW)

    strip = lax.slice(adj, (0, ncols_main), (n, n)).astype(bf16)
    if rem < SW:
        strip = jnp.pad(strip, ((0, 0), (0, SW - rem)))
    xb = x.astype(bf16)
    xp = jnp.pad(xb, ((0, n_pad - n), (0, 0))) if n_pad > n else xb
    s2_tail = n_pad - n

    b1r = b1.reshape(1, h1)
    b2r = b2.reshape(1, h2)
    w1b = W1.astype(bf16)
    w2b = W2.astype(bf16)
    lw = jnp.pad(lin_w, ((0, 0), (0, ncls_pad - ncls))).astype(bf16)
    lb = jnp.pad(lin_b, (0, ncls_pad - ncls)).reshape(1, ncls_pad)

    def body(adj_ref, strip_ref, x_ref, w1_ref, b1_ref, w2_ref, b2_ref,
             lw_ref, lb_ref, out_ref, s2_ref, oacc_ref, tacc_ref):
        p = pl.program_id(0)
        m = pl.program_id(1)
        k = pl.program_id(2)

        t = adj_ref[...].astype(bf16)
        mrows = pl.ds(m * bm, bm)
        krows = pl.ds(k * bk, bk)
        kf = (m * bm) // bk

        @pl.when(p == 0)
        def _phase0():
            prod = jnp.dot(t, x_ref[krows, :],
                           preferred_element_type=jnp.float32)

            @pl.when(k == 0)
            def _init():
                tacc_ref[...] = prod + jnp.dot(
                    strip_ref[...], x_ref[pl.ds(ncols_main, SW), :],
                    preferred_element_type=jnp.float32)
                oacc_ref[mrows, :] = jnp.zeros((bm, h2), jnp.float32)
                if s2_tail > 0:
                    @pl.when(m == 0)
                    def _zero_tail():
                        s2_ref[pl.ds(n, s2_tail), :] = jnp.zeros(
                            (s2_tail, h2), bf16)

            @pl.when(k > 0)
            def _acc():
                tacc_ref[...] += prod

            @pl.when(k < kf)
            def _dual():
                oacc_ref[mrows, :] += jnp.dot(
                    t, s2_ref[krows, :], preferred_element_type=jnp.float32)

            @pl.when(k == nk - 1)
            def _epilogue1():
                z = jnp.dot(tacc_ref[...].astype(bf16), w1_ref[...],
                            preferred_element_type=jnp.float32) + b1_ref[...]
                h = jnp.maximum(z, 0.1 * z)
                s2_ref[mrows, :] = jnp.dot(
                    h.astype(bf16), w2_ref[...],
                    preferred_element_type=jnp.float32).astype(bf16)

        @pl.when(p == 1)
        def _phase1():
            @pl.when(k >= kf)
            def _upper():
                oacc_ref[mrows, :] += jnp.dot(
                    t, s2_ref[krows, :], preferred_element_type=jnp.float32)

            @pl.when(k == nk - 1)
            def _epilogue2():
                z = oacc_ref[mrows, :] + jnp.dot(
                    strip_ref[...], s2_ref[pl.ds(ncols_main, SW), :],
                    preferred_element_type=jnp.float32) + b2_ref[...]
                h = jnp.maximum(z, 0.1 * z)
                out_ref[...] = jnp.dot(
                    h.astype(bf16), lw_ref[...],
                    preferred_element_type=jnp.float32) + lb_ref[...]

    def adj_index(p, m, k):
        kf = (m * bm) // bk
        return (m, jnp.where(p == 0, k, jnp.maximum(k, kf)))

    out = pl.pallas_call(
        body,
        grid=(2, nblk, nk),
        in_specs=[
            pl.BlockSpec((bm, bk), adj_index),
            pl.BlockSpec((bm, SW), lambda p, m, k: (m, 0)),
            pl.BlockSpec((n_pad, nfeat), lambda p, m, k: (0, 0)),
            pl.BlockSpec((nfeat, h1), lambda p, m, k: (0, 0)),
            pl.BlockSpec((1, h1), lambda p, m, k: (0, 0)),
            pl.BlockSpec((h1, h2), lambda p, m, k: (0, 0)),
            pl.BlockSpec((1, h2), lambda p, m, k: (0, 0)),
            pl.BlockSpec((h2, ncls_pad), lambda p, m, k: (0, 0)),
            pl.BlockSpec((1, ncls_pad), lambda p, m, k: (0, 0)),
        ],
        out_specs=pl.BlockSpec((bm, ncls_pad), lambda p, m, k: (p * m, 0)),
        out_shape=jax.ShapeDtypeStruct((n, ncls_pad), jnp.float32),
        scratch_shapes=[
            pltpu.VMEM((n_pad, h2), bf16),
            pltpu.VMEM((n, h2), jnp.float32),
            pltpu.VMEM((bm, nfeat), jnp.float32),
        ],
        compiler_params=pltpu.CompilerParams(
            dimension_semantics=("arbitrary", "arbitrary", "arbitrary"),
            vmem_limit_bytes=64 * 1024 * 1024,
        ),
    )(adj, strip, xp, W1, b1r, W2, b2r, lw, lb)

    return out[:, :ncls]
